# Initial kernel scaffold; baseline (speedup 1.0000x reference)
import jax
import jax.numpy as jnp
from jax.experimental import pallas as pl


def kernel(positions, table):
    raise NotImplementedError("write your pallas kernel here")


# speedup vs baseline: 2.1885x; 2.1885x over previous
import functools

import jax
import jax.numpy as jnp
from jax import lax
from jax.experimental import pallas as pl
from jax.experimental.pallas import tpu as pltpu
from jax.experimental.pallas import tpu_sc as plsc

DIM = 1024
CHUNK = 64


@functools.partial(jax.jit, static_argnames=("total",))
def _gather_sc(positions_flat, table, total):
    info = plsc.get_sparse_core_info()
    nc, ns = info.num_cores, info.num_subcores
    nw = nc * ns
    b_per_w = total // nw
    n_chunks = b_per_w // CHUNK
    mesh = plsc.VectorSubcoreMesh(core_axis_name="c", subcore_axis_name="s")

    @functools.partial(
        pl.kernel,
        mesh=mesh,
        out_type=jax.ShapeDtypeStruct((total, DIM), jnp.float32),
        scratch_types=[
            pltpu.VMEM((b_per_w,), jnp.int32),
            pltpu.VMEM((CHUNK, DIM), jnp.float32),
            pltpu.SemaphoreType.DMA,
        ],
    )
    def k(table_hbm, idx_hbm, out_hbm, idx_v, rows_v, sem):
        wid = lax.axis_index("s") * nc + lax.axis_index("c")
        base = wid * b_per_w
        pltpu.sync_copy(idx_hbm.at[pl.ds(base, b_per_w)], idx_v)

        def chunk_body(j, _):
            pltpu.async_copy(
                table_hbm.at[idx_v.at[pl.ds(j * CHUNK, CHUNK)]], rows_v, sem
            ).wait()
            pltpu.sync_copy(rows_v, out_hbm.at[pl.ds(base + j * CHUNK, CHUNK)])
            return 0

        lax.fori_loop(0, n_chunks, chunk_body, 0)

    return k(table, positions_flat)


def kernel(positions, table):
    b, s = positions.shape
    flat = positions.reshape(b * s).astype(jnp.int32)
    out = _gather_sc(flat, table, b * s)
    return out.reshape(b, s, DIM)


# speedup vs baseline: 2.2431x; 1.0250x over previous
import functools

import jax
import jax.numpy as jnp
from jax import lax
from jax.experimental import pallas as pl
from jax.experimental.pallas import tpu as pltpu
from jax.experimental.pallas import tpu_sc as plsc

DIM = 1024
CHUNK = 32


@functools.partial(jax.jit, static_argnames=("total",))
def _gather_sc(positions_flat, table, total):
    info = plsc.get_sparse_core_info()
    nc, ns = info.num_cores, info.num_subcores
    nw = nc * ns
    b_per_w = total // nw
    n_chunks = b_per_w // CHUNK
    n_pairs = n_chunks // 2
    mesh = plsc.VectorSubcoreMesh(core_axis_name="c", subcore_axis_name="s")

    @functools.partial(
        pl.kernel,
        mesh=mesh,
        out_type=jax.ShapeDtypeStruct((total, DIM), jnp.float32),
        scratch_types=[
            pltpu.VMEM((b_per_w,), jnp.int32),
            pltpu.VMEM((CHUNK, DIM), jnp.float32),
            pltpu.VMEM((CHUNK, DIM), jnp.float32),
            pltpu.SemaphoreType.DMA,
            pltpu.SemaphoreType.DMA,
            pltpu.SemaphoreType.DMA,
            pltpu.SemaphoreType.DMA,
        ],
    )
    def k(table_hbm, idx_hbm, out_hbm, idx_v, buf0, buf1, g0, g1, o0, o1):
        wid = lax.axis_index("s") * nc + lax.axis_index("c")
        base = wid * b_per_w
        pltpu.sync_copy(idx_hbm.at[pl.ds(base, b_per_w)], idx_v)

        bufs = (buf0, buf1)
        gsems = (g0, g1)
        osems = (o0, o1)

        def gather(i, b):
            return pltpu.make_async_copy(
                table_hbm.at[idx_v.at[pl.ds(i * CHUNK, CHUNK)]], bufs[b], gsems[b]
            )

        def put(i, b):
            return pltpu.make_async_copy(
                bufs[b], out_hbm.at[pl.ds(base + i * CHUNK, CHUNK)], osems[b]
            )

        gather(0, 0).start()
        gather(1, 1).start()

        def pair_body(p, _):
            i0 = 2 * p
            for b in range(2):
                i = i0 + b
                gather(i, b).wait()
                put(i, b).start()
            @pl.when(p + 1 < n_pairs)
            def _():
                for b in range(2):
                    i = i0 + b
                    put(i, b).wait()
                    gather(i + 2, b).start()

            return 0

        lax.fori_loop(0, n_pairs, pair_body, 0)
        put(n_chunks - 2, 0).wait()
        put(n_chunks - 1, 1).wait()

    return k(table, positions_flat)


def kernel(positions, table):
    b, s = positions.shape
    flat = positions.reshape(b * s).astype(jnp.int32)
    out = _gather_sc(flat, table, b * s)
    return out.reshape(b, s, DIM)


# speedup vs baseline: 2.3022x; 1.0263x over previous
import functools

import jax
import jax.numpy as jnp
from jax import lax
from jax.experimental import pallas as pl
from jax.experimental.pallas import tpu as pltpu
from jax.experimental.pallas import tpu_sc as plsc

DIM = 1024
CHUNK = 16


@functools.partial(jax.jit, static_argnames=("total",))
def _gather_sc(positions_flat, table, total):
    info = plsc.get_sparse_core_info()
    nc, ns = info.num_cores, info.num_subcores
    nw = nc * ns
    b_per_w = total // nw
    n_chunks = b_per_w // CHUNK
    n_buf = 4
    n_rounds = n_chunks // n_buf
    mesh = plsc.VectorSubcoreMesh(core_axis_name="c", subcore_axis_name="s")

    @functools.partial(
        pl.kernel,
        mesh=mesh,
        out_type=jax.ShapeDtypeStruct((total, DIM), jnp.float32),
        scratch_types=[
            pltpu.VMEM((b_per_w,), jnp.int32),
            pltpu.VMEM((CHUNK, DIM), jnp.float32),
            pltpu.VMEM((CHUNK, DIM), jnp.float32),
            pltpu.VMEM((CHUNK, DIM), jnp.float32),
            pltpu.VMEM((CHUNK, DIM), jnp.float32),
            pltpu.SemaphoreType.DMA,
            pltpu.SemaphoreType.DMA,
            pltpu.SemaphoreType.DMA,
            pltpu.SemaphoreType.DMA,
            pltpu.SemaphoreType.DMA,
            pltpu.SemaphoreType.DMA,
            pltpu.SemaphoreType.DMA,
            pltpu.SemaphoreType.DMA,
        ],
    )
    def k(table_hbm, idx_hbm, out_hbm, idx_v, buf0, buf1, buf2, buf3,
          g0, g1, g2, g3, o0, o1, o2, o3):
        wid = lax.axis_index("s") * nc + lax.axis_index("c")
        base = wid * b_per_w
        pltpu.sync_copy(idx_hbm.at[pl.ds(base, b_per_w)], idx_v)

        bufs = (buf0, buf1, buf2, buf3)
        gsems = (g0, g1, g2, g3)
        osems = (o0, o1, o2, o3)

        def gather(i, b):
            return pltpu.make_async_copy(
                table_hbm.at[idx_v.at[pl.ds(i * CHUNK, CHUNK)]], bufs[b], gsems[b]
            )

        def put(i, b):
            return pltpu.make_async_copy(
                bufs[b], out_hbm.at[pl.ds(base + i * CHUNK, CHUNK)], osems[b]
            )

        for b in range(n_buf):
            gather(b, b).start()

        def round_body(p, _):
            i0 = p * n_buf
            for b in range(n_buf):
                i = i0 + b
                gather(i, b).wait()
                put(i, b).start()
            @pl.when(p + 1 < n_rounds)
            def _():
                for b in range(n_buf):
                    i = i0 + b
                    put(i, b).wait()
                    gather(i + n_buf, b).start()

            return 0

        lax.fori_loop(0, n_rounds, round_body, 0)
        for b in range(n_buf):
            put(n_chunks - n_buf + b, b).wait()

    return k(table, positions_flat)


def kernel(positions, table):
    b, s = positions.shape
    flat = positions.reshape(b * s).astype(jnp.int32)
    out = _gather_sc(flat, table, b * s)
    return out.reshape(b, s, DIM)
